# Initial kernel scaffold; baseline (speedup 1.0000x reference)
import jax
import jax.numpy as jnp
from jax.experimental import pallas as pl


def kernel(token_ids, word_lut, pe_table):
    raise NotImplementedError("write your pallas kernel here")


# speedup vs baseline: 1.3446x; 1.3446x over previous
import functools
import math

import jax
import jax.numpy as jnp
from jax.experimental import pallas as pl
from jax.experimental.pallas import tpu as pltpu


def _gather_embed_kernel(ids_ref, table_hbm, pe_ref, out_ref, gbuf, sem,
                         *, scale, tile_len, batch, n_inner):
    c = pl.program_id(0)
    j = pl.program_id(1)
    slot = jax.lax.rem(j, 2)
    rows = tile_len * batch

    def issue(tile_idx, dst_slot):
        base = tile_idx * rows
        for s in range(tile_len):
            for b in range(batch):
                tok = ids_ref[base + s * batch + b]
                pltpu.make_async_copy(
                    table_hbm.at[tok],
                    gbuf.at[dst_slot, s, b],
                    sem.at[dst_slot],
                ).start()

    @pl.when(j == 0)
    def _():
        issue(c * n_inner, slot)

    @pl.when(j + 1 < n_inner)
    def _():
        issue(c * n_inner + j + 1, 1 - slot)

    pltpu.make_async_copy(gbuf.at[slot], gbuf.at[slot], sem.at[slot]).wait()

    out_ref[...] = gbuf[slot] * scale + pe_ref[...]


def kernel(token_ids, word_lut, pe_table):
    seq_len, batch, nfeat = token_ids.shape
    assert nfeat == 1
    vocab, dim = word_lut.shape
    scale = float(math.sqrt(dim))

    tile_len = 16
    n_cores = 2
    n_inner = seq_len // tile_len // n_cores

    ids_flat = token_ids[:, :, 0].reshape(seq_len * batch).astype(jnp.int32)
    pe3 = pe_table[:seq_len].reshape(seq_len, 1, dim)

    body = functools.partial(
        _gather_embed_kernel,
        scale=scale, tile_len=tile_len, batch=batch, n_inner=n_inner,
    )

    grid_spec = pltpu.PrefetchScalarGridSpec(
        num_scalar_prefetch=1,
        grid=(n_cores, n_inner),
        in_specs=[
            pl.BlockSpec(memory_space=pl.ANY),
            pl.BlockSpec((tile_len, 1, dim),
                         lambda c, j, ids: (c * n_inner + j, 0, 0)),
        ],
        out_specs=pl.BlockSpec((tile_len, batch, dim),
                               lambda c, j, ids: (c * n_inner + j, 0, 0)),
        scratch_shapes=[
            pltpu.VMEM((2, tile_len, batch, dim), word_lut.dtype),
            pltpu.SemaphoreType.DMA((2,)),
        ],
    )

    out = pl.pallas_call(
        body,
        grid_spec=grid_spec,
        out_shape=jax.ShapeDtypeStruct((seq_len, batch, dim), word_lut.dtype),
        compiler_params=pltpu.CompilerParams(
            dimension_semantics=("parallel", "arbitrary"),
            disable_bounds_checks=True,
        ),
    )(ids_flat, word_lut, pe3)
    return out


# speedup vs baseline: 1.3637x; 1.0142x over previous
import functools
import math

import jax
import jax.numpy as jnp
from jax.experimental import pallas as pl
from jax.experimental.pallas import tpu as pltpu


def _gather_embed_kernel(ids_ref, table_hbm, pe_ref, out_ref, gbuf, sem,
                         *, scale, tile_len, batch, n_inner):
    c = pl.program_id(0)
    j = pl.program_id(1)
    slot = jax.lax.rem(j, 2)
    rows = tile_len * batch

    def issue(tile_idx, dst_slot):
        base = tile_idx * rows
        for s in range(tile_len):
            for b in range(batch):
                tok = ids_ref[base + s * batch + b]
                pltpu.make_async_copy(
                    table_hbm.at[tok],
                    gbuf.at[dst_slot, s, b],
                    sem.at[dst_slot],
                ).start()

    @pl.when(j == 0)
    def _():
        issue(c * n_inner, slot)

    @pl.when(j + 1 < n_inner)
    def _():
        issue(c * n_inner + j + 1, 1 - slot)

    pltpu.make_async_copy(gbuf.at[slot], gbuf.at[slot], sem.at[slot]).wait()

    out_ref[...] = gbuf[slot] * scale + pe_ref[...]


def kernel(token_ids, word_lut, pe_table):
    seq_len, batch, nfeat = token_ids.shape
    assert nfeat == 1
    vocab, dim = word_lut.shape
    scale = float(math.sqrt(dim))

    tile_len = 32
    n_cores = 2
    n_inner = seq_len // tile_len // n_cores

    ids_flat = token_ids[:, :, 0].reshape(seq_len * batch).astype(jnp.int32)
    pe3 = pe_table[:seq_len].reshape(seq_len, 1, dim)

    body = functools.partial(
        _gather_embed_kernel,
        scale=scale, tile_len=tile_len, batch=batch, n_inner=n_inner,
    )

    grid_spec = pltpu.PrefetchScalarGridSpec(
        num_scalar_prefetch=1,
        grid=(n_cores, n_inner),
        in_specs=[
            pl.BlockSpec(memory_space=pl.ANY),
            pl.BlockSpec((tile_len, 1, dim),
                         lambda c, j, ids: (c * n_inner + j, 0, 0)),
        ],
        out_specs=pl.BlockSpec((tile_len, batch, dim),
                               lambda c, j, ids: (c * n_inner + j, 0, 0)),
        scratch_shapes=[
            pltpu.VMEM((2, tile_len, batch, dim), word_lut.dtype),
            pltpu.SemaphoreType.DMA((2,)),
        ],
    )

    out = pl.pallas_call(
        body,
        grid_spec=grid_spec,
        out_shape=jax.ShapeDtypeStruct((seq_len, batch, dim), word_lut.dtype),
        compiler_params=pltpu.CompilerParams(
            dimension_semantics=("parallel", "arbitrary"),
            disable_bounds_checks=True,
        ),
    )(ids_flat, word_lut, pe3)
    return out
